# baseline (speedup 1.0000x reference)
import jax
import jax.numpy as jnp
from jax import lax
from jax.experimental import pallas as pl
from jax.experimental.pallas import tpu as pltpu
from jax.experimental.pallas import tpu_sc as plsc

N_FACTORS = 64
BATCH = 16384
NC = 2
NS = 16
NW = NC * NS
B_PER_W = BATCH // NW
CHUNK = 128
N_CHUNKS = B_PER_W // CHUNK
LANES = 16


def _body(user_ref, item_ref, uf_ref, if_ref, out_ref,
          idxb, ubuf, vbuf, tbuf, outv, sem):
    wid = lax.axis_index("s") * NC + lax.axis_index("c")
    base = wid * B_PER_W

    for j in range(N_CHUNKS):
        pltpu.sync_copy(user_ref.at[wid, j], idxb.at[j])
        pltpu.sync_copy(item_ref.at[wid, j], idxb.at[N_CHUNKS + j])
        pltpu.sync_copy(item_ref.at[wid, j],
                        idxb.at[2 * (N_CHUNKS + j)])

    lane_iota = lax.iota(jnp.int32, LANES)

    for j in range(N_CHUNKS):
        cu = pltpu.async_copy(uf_ref.at[idxb.at[2 * j]], ubuf, sem)
        cv = pltpu.async_copy(
            if_ref.at[idxb.at[2 * (N_CHUNKS + j)]], vbuf, sem)
        cu.wait()
        cv.wait()

        def group(g, carry):
            for i in range(LANES):
                r = g * LANES + i
                s = jnp.zeros((LANES,), jnp.float32)
                for c in range(N_FACTORS // LANES):
                    u = ubuf[r, 0, pl.ds(c * LANES, LANES)]
                    v = vbuf[r, 0, pl.ds(c * LANES, LANES)]
                    s = s + u * v
                plsc.store_scatter(tbuf, [lane_iota * LANES + i], s)
            pl.delay(100)
            acc = jnp.zeros((LANES,), jnp.float32)
            for l in range(LANES):
                acc = acc + tbuf[pl.ds(l * LANES, LANES)]
            outv[pl.ds(j * CHUNK + g * LANES, LANES)] = acc
            return carry

        lax.fori_loop(0, CHUNK // LANES, group, 0)

    pltpu.sync_copy(outv, out_ref.at[pl.ds(base, B_PER_W)])


@jax.jit
def kernel(user, item, user_factors, item_factors):
    user = user.astype(jnp.int32).reshape(NW, N_CHUNKS, CHUNK)
    item = item.astype(jnp.int32).reshape(NW, N_CHUNKS, CHUNK)
    user_factors = user_factors.reshape(user_factors.shape[0], 1, N_FACTORS)
    item_factors = item_factors.reshape(item_factors.shape[0], 1, N_FACTORS)
    mesh = plsc.VectorSubcoreMesh(core_axis_name="c", subcore_axis_name="s")
    k = pl.kernel(
        _body,
        out_type=jax.ShapeDtypeStruct((BATCH,), jnp.float32),
        mesh=mesh,
        scratch_types=[
            pltpu.VMEM((4 * N_CHUNKS, CHUNK), jnp.int32),
            pltpu.VMEM((CHUNK, 1, N_FACTORS), jnp.float32),
            pltpu.VMEM((CHUNK, 1, N_FACTORS), jnp.float32),
            pltpu.VMEM((LANES * LANES,), jnp.float32),
            pltpu.VMEM((B_PER_W,), jnp.float32),
            pltpu.SemaphoreType.DMA,
        ],
        compiler_params=pltpu.CompilerParams(needs_layout_passes=False),
    )
    return k(user, item, user_factors, item_factors)


# speedup vs baseline: 4.0407x; 4.0407x over previous
import jax
import jax.numpy as jnp
from jax import lax
from jax.experimental import layout
from jax.experimental import pallas as pl
from jax.experimental.pallas import tpu as pltpu
from jax.experimental.pallas import tpu_sc as plsc

N_FACTORS = 64
BATCH = 16384
NC = 2
NS = 16
NW = NC * NS
B_PER_W = BATCH // NW
CHUNK = 128
N_CHUNKS = B_PER_W // CHUNK
LANES = 16


def _body(user_ref, item_ref, uf_ref, if_ref, out_ref,
          uidxv, iidxv, ubuf, vbuf, tbuf, outv, sem):
    wid = lax.axis_index("s") * NC + lax.axis_index("c")
    base = wid * B_PER_W

    pltpu.sync_copy(user_ref.at[wid], uidxv)
    pltpu.sync_copy(item_ref.at[wid], iidxv)

    lane_iota = lax.iota(jnp.int32, LANES)

    for j in range(N_CHUNKS):

        def fetch_group(g, carry, j=j):
            uvec = uidxv[pl.ds(j * CHUNK + g * LANES, LANES)]
            ivec = iidxv[pl.ds(j * CHUNK + g * LANES, LANES)]
            descs = []
            for l in range(LANES):
                r = g * LANES + l
                descs.append(pltpu.async_copy(
                    uf_ref.at[pl.ds(uvec[l], 1), :],
                    ubuf.at[pl.ds(r, 1), :], sem))
                descs.append(pltpu.async_copy(
                    if_ref.at[pl.ds(ivec[l], 1), :],
                    vbuf.at[pl.ds(r, 1), :], sem))
            for c in descs:
                c.wait()
            return carry

        lax.fori_loop(0, CHUNK // LANES, fetch_group, 0)

        def dot_group(g, carry, j=j):
            for i in range(LANES):
                r = g * LANES + i
                s = jnp.zeros((LANES,), jnp.float32)
                for c in range(N_FACTORS // LANES):
                    u = ubuf[r, pl.ds(c * LANES, LANES)]
                    v = vbuf[r, pl.ds(c * LANES, LANES)]
                    s = s + u * v
                plsc.store_scatter(tbuf, [lane_iota * LANES + i], s)
            pl.delay(100)
            acc = jnp.zeros((LANES,), jnp.float32)
            for l in range(LANES):
                acc = acc + tbuf[pl.ds(l * LANES, LANES)]
            outv[pl.ds(j * CHUNK + g * LANES, LANES)] = acc
            return carry

        lax.fori_loop(0, CHUNK // LANES, dot_group, 0)

    pltpu.sync_copy(outv, out_ref.at[pl.ds(base, B_PER_W)])


@jax.jit
def kernel(user, item, user_factors, item_factors):
    user = user.astype(jnp.int32).reshape(NW, B_PER_W)
    item = item.astype(jnp.int32).reshape(NW, B_PER_W)
    rm = layout.Layout((0, 1))
    user_factors = layout.with_layout_constraint(user_factors, rm)
    item_factors = layout.with_layout_constraint(item_factors, rm)
    user_factors, item_factors = lax.optimization_barrier(
        (user_factors, item_factors))
    mesh = plsc.VectorSubcoreMesh(core_axis_name="c", subcore_axis_name="s")
    k = pl.kernel(
        _body,
        out_type=jax.ShapeDtypeStruct((BATCH,), jnp.float32),
        mesh=mesh,
        scratch_types=[
            pltpu.VMEM((B_PER_W,), jnp.int32),
            pltpu.VMEM((B_PER_W,), jnp.int32),
            pltpu.VMEM((CHUNK, N_FACTORS), jnp.float32),
            pltpu.VMEM((CHUNK, N_FACTORS), jnp.float32),
            pltpu.VMEM((LANES * LANES,), jnp.float32),
            pltpu.VMEM((B_PER_W,), jnp.float32),
            pltpu.SemaphoreType.DMA,
        ],
        compiler_params=pltpu.CompilerParams(needs_layout_passes=False),
    )
    return k(user, item, user_factors, item_factors)


# speedup vs baseline: 4.1418x; 1.0250x over previous
import jax
import jax.numpy as jnp
from jax import lax
from jax.experimental import layout
from jax.experimental import pallas as pl
from jax.experimental.pallas import tpu as pltpu
from jax.experimental.pallas import tpu_sc as plsc

N_FACTORS = 64
BATCH = 16384
NC = 2
NS = 16
NW = NC * NS
B_PER_W = BATCH // NW
CHUNK = 128
N_CHUNKS = B_PER_W // CHUNK
LANES = 16


def _body(user_ref, item_ref, uf_ref, if_ref, out_ref,
          uidxv, iidxv, ubuf, vbuf, tbuf, outv, sem):
    wid = lax.axis_index("s") * NC + lax.axis_index("c")
    base = wid * B_PER_W

    pltpu.sync_copy(user_ref.at[wid], uidxv)
    pltpu.sync_copy(item_ref.at[wid], iidxv)

    lane_iota = lax.iota(jnp.int32, LANES)

    for j in range(N_CHUNKS):

        def fetch_group(g, carry, j=j):
            descs = []
            for q in range(4):
                uvec = uidxv[pl.ds(j * CHUNK + g * 64 + q * LANES, LANES)]
                ivec = iidxv[pl.ds(j * CHUNK + g * 64 + q * LANES, LANES)]
                for l in range(LANES):
                    r = g * 64 + q * LANES + l
                    descs.append(pltpu.async_copy(
                        uf_ref.at[pl.ds(uvec[l], 1), :],
                        ubuf.at[pl.ds(r, 1), :], sem))
                    descs.append(pltpu.async_copy(
                        if_ref.at[pl.ds(ivec[l], 1), :],
                        vbuf.at[pl.ds(r, 1), :], sem))
            for c in descs:
                c.wait()
            return carry

        lax.fori_loop(0, CHUNK // 64, fetch_group, 0)

        def dot_group(g, carry, j=j):
            for i in range(LANES):
                r = g * LANES + i
                s = jnp.zeros((LANES,), jnp.float32)
                for c in range(N_FACTORS // LANES):
                    u = ubuf[r, pl.ds(c * LANES, LANES)]
                    v = vbuf[r, pl.ds(c * LANES, LANES)]
                    s = s + u * v
                plsc.store_scatter(tbuf, [lane_iota * LANES + i], s)
            pl.delay(20)
            acc = jnp.zeros((LANES,), jnp.float32)
            for l in range(LANES):
                acc = acc + tbuf[pl.ds(l * LANES, LANES)]
            outv[pl.ds(j * CHUNK + g * LANES, LANES)] = acc
            return carry

        lax.fori_loop(0, CHUNK // LANES, dot_group, 0)

    pltpu.sync_copy(outv, out_ref.at[pl.ds(base, B_PER_W)])


@jax.jit
def kernel(user, item, user_factors, item_factors):
    user = user.astype(jnp.int32).reshape(NW, B_PER_W)
    item = item.astype(jnp.int32).reshape(NW, B_PER_W)
    rm = layout.Layout((0, 1))
    user_factors = layout.with_layout_constraint(user_factors, rm)
    item_factors = layout.with_layout_constraint(item_factors, rm)
    user_factors, item_factors = lax.optimization_barrier(
        (user_factors, item_factors))
    mesh = plsc.VectorSubcoreMesh(core_axis_name="c", subcore_axis_name="s")
    k = pl.kernel(
        _body,
        out_type=jax.ShapeDtypeStruct((BATCH,), jnp.float32),
        mesh=mesh,
        scratch_types=[
            pltpu.VMEM((B_PER_W,), jnp.int32),
            pltpu.VMEM((B_PER_W,), jnp.int32),
            pltpu.VMEM((CHUNK, N_FACTORS), jnp.float32),
            pltpu.VMEM((CHUNK, N_FACTORS), jnp.float32),
            pltpu.VMEM((LANES * LANES,), jnp.float32),
            pltpu.VMEM((B_PER_W,), jnp.float32),
            pltpu.SemaphoreType.DMA,
        ],
        compiler_params=pltpu.CompilerParams(needs_layout_passes=False),
    )
    return k(user, item, user_factors, item_factors)


# speedup vs baseline: 4.1670x; 1.0061x over previous
import jax
import jax.numpy as jnp
from jax import lax
from jax.experimental import layout
from jax.experimental import pallas as pl
from jax.experimental.pallas import tpu as pltpu
from jax.experimental.pallas import tpu_sc as plsc

N_FACTORS = 64
BATCH = 16384
NC = 2
NS = 16
NW = NC * NS
B_PER_W = BATCH // NW
CHUNK = 128
N_CHUNKS = B_PER_W // CHUNK
LANES = 16


def _body(user_ref, item_ref, uf_ref, if_ref, out_ref,
          uidxv, iidxv, ubuf0, ubuf1, vbuf0, vbuf1, tbuf, outv, sem):
    wid = lax.axis_index("s") * NC + lax.axis_index("c")
    base = wid * B_PER_W

    pltpu.sync_copy(user_ref.at[wid], uidxv)
    pltpu.sync_copy(item_ref.at[wid], iidxv)

    lane_iota = lax.iota(jnp.int32, LANES)

    ubufs = [ubuf0, ubuf1]
    vbufs = [vbuf0, vbuf1]

    def fire(j):
        descs = []
        ub = ubufs[j % 2]
        vb = vbufs[j % 2]
        for q in range(CHUNK // LANES):
            uvec = uidxv[pl.ds(j * CHUNK + q * LANES, LANES)]
            ivec = iidxv[pl.ds(j * CHUNK + q * LANES, LANES)]
            for l in range(LANES):
                r = q * LANES + l
                pltpu.async_copy(
                    uf_ref.at[pl.ds(uvec[l], 1), :],
                    ub.at[pl.ds(r, 1), :], sem)
                pltpu.async_copy(
                    if_ref.at[pl.ds(ivec[l], 1), :],
                    vb.at[pl.ds(r, 1), :], sem)
        descs.append(pltpu.make_async_copy(
            uf_ref.at[pl.ds(0, CHUNK), :], ub, sem))
        descs.append(pltpu.make_async_copy(
            if_ref.at[pl.ds(0, CHUNK), :], vb, sem))
        return descs

    def compute(j):
        ub = ubufs[j % 2]
        vb = vbufs[j % 2]

        def dot_group(g, carry):
            for i in range(LANES):
                r = g * LANES + i
                s = jnp.zeros((LANES,), jnp.float32)
                for c in range(N_FACTORS // LANES):
                    u = ub[r, pl.ds(c * LANES, LANES)]
                    v = vb[r, pl.ds(c * LANES, LANES)]
                    s = s + u * v
                plsc.store_scatter(tbuf, [lane_iota * LANES + i], s)
            pl.delay(20)
            acc = jnp.zeros((LANES,), jnp.float32)
            for l in range(LANES):
                acc = acc + tbuf[pl.ds(l * LANES, LANES)]
            outv[pl.ds(j * CHUNK + g * LANES, LANES)] = acc
            return carry

        lax.fori_loop(0, CHUNK // LANES, dot_group, 0)

    descs = fire(0)
    for j in range(N_CHUNKS):
        nxt = fire(j + 1) if j + 1 < N_CHUNKS else []
        for c in descs:
            c.wait()
        compute(j)
        descs = nxt

    pltpu.sync_copy(outv, out_ref.at[pl.ds(base, B_PER_W)])


@jax.jit
def kernel(user, item, user_factors, item_factors):
    user = user.astype(jnp.int32).reshape(NW, B_PER_W)
    item = item.astype(jnp.int32).reshape(NW, B_PER_W)
    rm = layout.Layout((0, 1))
    user_factors = layout.with_layout_constraint(user_factors, rm)
    item_factors = layout.with_layout_constraint(item_factors, rm)
    user_factors, item_factors = lax.optimization_barrier(
        (user_factors, item_factors))
    mesh = plsc.VectorSubcoreMesh(core_axis_name="c", subcore_axis_name="s")
    k = pl.kernel(
        _body,
        out_type=jax.ShapeDtypeStruct((BATCH,), jnp.float32),
        mesh=mesh,
        scratch_types=[
            pltpu.VMEM((B_PER_W,), jnp.int32),
            pltpu.VMEM((B_PER_W,), jnp.int32),
            pltpu.VMEM((CHUNK, N_FACTORS), jnp.float32),
            pltpu.VMEM((CHUNK, N_FACTORS), jnp.float32),
            pltpu.VMEM((CHUNK, N_FACTORS), jnp.float32),
            pltpu.VMEM((CHUNK, N_FACTORS), jnp.float32),
            pltpu.VMEM((LANES * LANES,), jnp.float32),
            pltpu.VMEM((B_PER_W,), jnp.float32),
            pltpu.SemaphoreType.DMA,
        ],
        compiler_params=pltpu.CompilerParams(needs_layout_passes=False),
    )
    return k(user, item, user_factors, item_factors)


# speedup vs baseline: 4.1771x; 1.0024x over previous
import jax
import jax.numpy as jnp
from jax import lax
from jax.experimental import layout
from jax.experimental import pallas as pl
from jax.experimental.pallas import tpu as pltpu
from jax.experimental.pallas import tpu_sc as plsc

N_FACTORS = 64
BATCH = 16384
NC = 2
NS = 16
NW = NC * NS
B_PER_W = BATCH // NW
CHUNK = 128
N_CHUNKS = B_PER_W // CHUNK
LANES = 16


def _body(user_ref, item_ref, uf_ref, if_ref, out_ref,
          uidxv, iidxv, ubuf0, ubuf1, vbuf0, vbuf1, tbuf, outv,
          sem, semu0, semu1, semv0, semv1):
    wid = lax.axis_index("s") * NC + lax.axis_index("c")
    base = wid * B_PER_W

    pltpu.sync_copy(user_ref.at[wid], uidxv)
    pltpu.sync_copy(item_ref.at[wid], iidxv)

    lane_iota = lax.iota(jnp.int32, LANES)

    ubufs = [ubuf0, ubuf1]
    vbufs = [vbuf0, vbuf1]

    def fire(j):
        descs = []
        ub = ubufs[j % 2]
        vb = vbufs[j % 2]
        for q in range(CHUNK // LANES):
            uvec = uidxv[pl.ds(j * CHUNK + q * LANES, LANES)]
            ivec = iidxv[pl.ds(j * CHUNK + q * LANES, LANES)]
            for l in range(LANES):
                r = q * LANES + l
                pltpu.async_copy(
                    uf_ref.at[pl.ds(uvec[l], 1), :],
                    ub.at[pl.ds(r, 1), :], semu0 if r % 2 else semu1)
                pltpu.async_copy(
                    if_ref.at[pl.ds(ivec[l], 1), :],
                    vb.at[pl.ds(r, 1), :], semv0 if r % 2 else semv1)
        half = uf_ref.at[pl.ds(0, CHUNK // 2), :]
        descs.append(pltpu.make_async_copy(
            half, ub.at[pl.ds(0, CHUNK // 2), :], semu0))
        descs.append(pltpu.make_async_copy(
            half, ub.at[pl.ds(0, CHUNK // 2), :], semu1))
        descs.append(pltpu.make_async_copy(
            half, vb.at[pl.ds(0, CHUNK // 2), :], semv0))
        descs.append(pltpu.make_async_copy(
            half, vb.at[pl.ds(0, CHUNK // 2), :], semv1))
        return descs

    def compute(j):
        ub = ubufs[j % 2]
        vb = vbufs[j % 2]

        def dot_group(g, carry):
            for i in range(LANES):
                r = g * LANES + i
                s = jnp.zeros((LANES,), jnp.float32)
                for c in range(N_FACTORS // LANES):
                    u = ub[r, pl.ds(c * LANES, LANES)]
                    v = vb[r, pl.ds(c * LANES, LANES)]
                    s = s + u * v
                plsc.store_scatter(tbuf, [lane_iota * LANES + i], s)
            pl.delay(20)
            acc = jnp.zeros((LANES,), jnp.float32)
            for l in range(LANES):
                acc = acc + tbuf[pl.ds(l * LANES, LANES)]
            outv[pl.ds(j * CHUNK + g * LANES, LANES)] = acc
            return carry

        lax.fori_loop(0, CHUNK // LANES, dot_group, 0)

    descs = fire(0)
    for j in range(N_CHUNKS):
        nxt = fire(j + 1) if j + 1 < N_CHUNKS else []
        for c in descs:
            c.wait()
        compute(j)
        descs = nxt

    pltpu.sync_copy(outv, out_ref.at[pl.ds(base, B_PER_W)])


@jax.jit
def kernel(user, item, user_factors, item_factors):
    user = user.astype(jnp.int32).reshape(NW, B_PER_W)
    item = item.astype(jnp.int32).reshape(NW, B_PER_W)
    rm = layout.Layout((0, 1))
    user_factors = layout.with_layout_constraint(user_factors, rm)
    item_factors = layout.with_layout_constraint(item_factors, rm)
    user_factors, item_factors = lax.optimization_barrier(
        (user_factors, item_factors))
    mesh = plsc.VectorSubcoreMesh(core_axis_name="c", subcore_axis_name="s")
    k = pl.kernel(
        _body,
        out_type=jax.ShapeDtypeStruct((BATCH,), jnp.float32),
        mesh=mesh,
        scratch_types=[
            pltpu.VMEM((B_PER_W,), jnp.int32),
            pltpu.VMEM((B_PER_W,), jnp.int32),
            pltpu.VMEM((CHUNK, N_FACTORS), jnp.float32),
            pltpu.VMEM((CHUNK, N_FACTORS), jnp.float32),
            pltpu.VMEM((CHUNK, N_FACTORS), jnp.float32),
            pltpu.VMEM((CHUNK, N_FACTORS), jnp.float32),
            pltpu.VMEM((LANES * LANES,), jnp.float32),
            pltpu.VMEM((B_PER_W,), jnp.float32),
            pltpu.SemaphoreType.DMA,
            pltpu.SemaphoreType.DMA,
            pltpu.SemaphoreType.DMA,
            pltpu.SemaphoreType.DMA,
            pltpu.SemaphoreType.DMA,
        ],
        compiler_params=pltpu.CompilerParams(needs_layout_passes=False),
    )
    return k(user, item, user_factors, item_factors)
